# baseline (speedup 1.0000x reference)
import functools
import math

import jax
import jax.numpy as jnp
from jax import lax
from jax.experimental import pallas as pl
from jax.experimental.pallas import tpu as pltpu
from jax.experimental.pallas import tpu_sc as plsc

TB = 512
BLK = 512


def _gelu_exact(h):
    return h * 0.5 * (1.0 + lax.erf(h * (1.0 / math.sqrt(2.0))))


def _router_body(ridx_ref, z_ref, c_ref, tau_ref,
                 d0_ref, d1_ref, rw0_ref, rw1_ref, ew_ref,
                 bexp_ref, brow_ref, bact_ref, carry_ref,
                 *, E, B, CAP, RPB, MAXB, NB):
    i = pl.program_id(0)
    zb = z_ref[...]
    c = c_ref[0]

    t = tau_ref[ridx_ref[0], 0]
    tau = jnp.maximum(t, 0.0) + jnp.log1p(jnp.exp(-jnp.abs(t))) + 1e-6
    d2_cols = []
    for e in range(E):
        diff = zb - c[e:e + 1, :]
        d2_cols.append(jnp.sum(diff * diff, axis=1, keepdims=True))
    dist = jnp.sqrt(jnp.concatenate(d2_cols, axis=1))
    scores = jnp.exp(-dist / tau)

    e_iota = lax.broadcasted_iota(jnp.int32, scores.shape, 1)
    s0 = jnp.max(scores, axis=1, keepdims=True)
    i0 = jnp.argmax(scores, axis=1)[:, None]
    masked = jnp.where(e_iota == i0, -jnp.inf, scores)
    s1 = jnp.max(masked, axis=1, keepdims=True)
    i1 = jnp.argmax(masked, axis=1)[:, None]
    w0 = 1.0 / (1.0 + jnp.exp(s1 - s0))
    rw0_ref[...] = w0
    rw1_ref[...] = 1.0 - w0

    onehot0 = (i0 == e_iota).astype(jnp.float32)
    onehot1 = (i1 == e_iota).astype(jnp.float32)
    cnt = onehot0 + onehot1

    @pl.when(i == 0)
    def _():
        carry_ref[...] = jnp.zeros_like(carry_ref)
        ew_ref[...] = jnp.zeros_like(ew_ref)

    base = carry_ref[...]
    r_iota = lax.broadcasted_iota(jnp.int32, (TB, TB), 0)
    c_iota = lax.broadcasted_iota(jnp.int32, (TB, TB), 1)
    tri = (c_iota < r_iota).astype(jnp.float32)
    pre = lax.dot_general(tri, cnt, (((1,), (0,)), ((), ())),
                          preferred_element_type=jnp.float32)
    pre = pre + base
    rank0 = jnp.sum(pre * onehot0, axis=1, keepdims=True)
    rank1 = jnp.sum(pre * onehot1, axis=1, keepdims=True)
    d0_ref[...] = i0 * CAP + rank0.astype(jnp.int32)
    d1_ref[...] = i1 * CAP + rank1.astype(jnp.int32)

    tot = jnp.sum(cnt, axis=0, keepdims=True)
    new_carry = base + tot
    carry_ref[...] = new_carry
    ew_ref[...] += tot / (B * 2.0)

    @pl.when(i == NB - 1)
    def _():
        counts = new_carry.astype(jnp.int32)
        nb = lax.shift_right_logical(counts + (BLK - 1),
                                     int(math.log2(BLK)))
        er = lax.broadcasted_iota(jnp.int32, (E, E), 0)
        ec = lax.broadcasted_iota(jnp.int32, (E, E), 1)
        tri_e = (er < ec).astype(jnp.float32)
        cumx = lax.dot_general(nb.astype(jnp.float32), tri_e,
                               (((1,), (0,)), ((), ())),
                               preferred_element_type=jnp.float32)
        cumx = cumx.astype(jnp.int32)
        total_nb = cumx[0, E - 1] + nb[0, E - 1]
        biota = lax.broadcasted_iota(jnp.int32, (1, MAXB), 1)
        eb = jnp.zeros((1, MAXB), jnp.int32)
        cumsel = jnp.zeros((1, MAXB), jnp.int32)
        for e in range(E):
            ge = (biota >= cumx[0, e]).astype(jnp.int32)
            eb = eb + ge
            if e > 0:
                cumsel = jnp.where(eb == e + 1, cumx[0, e], cumsel)
        eb = eb - 1
        act = (biota < total_nb).astype(jnp.int32)
        rowblk = eb * RPB + (biota - cumsel)
        rowblk = jnp.where(act == 1, rowblk, E * RPB)
        bexp_ref[...] = jnp.where(act == 1, eb, E - 1)
        brow_ref[...] = rowblk
        bact_ref[...] = act


def _run_router(z, router_idx, centroids, tau_raw, E, B, D, R,
                CAP, RPB, MAXB):
    NB = B // TB
    ridx = jnp.asarray(router_idx, jnp.int32).reshape((1,))
    body = functools.partial(_router_body, E=E, B=B, CAP=CAP, RPB=RPB,
                             MAXB=MAXB, NB=NB)
    grid_spec = pltpu.PrefetchScalarGridSpec(
        num_scalar_prefetch=1,
        grid=(NB,),
        in_specs=[
            pl.BlockSpec((TB, D), lambda i, r: (i, 0)),
            pl.BlockSpec((1, E, D), lambda i, r: (r[0], 0, 0)),
            pl.BlockSpec(memory_space=pltpu.SMEM),
        ],
        out_specs=[
            pl.BlockSpec((TB, 1), lambda i, r: (i, 0)),
            pl.BlockSpec((TB, 1), lambda i, r: (i, 0)),
            pl.BlockSpec((TB, 1), lambda i, r: (i, 0)),
            pl.BlockSpec((TB, 1), lambda i, r: (i, 0)),
            pl.BlockSpec((1, E), lambda i, r: (0, 0)),
            pl.BlockSpec((1, MAXB), lambda i, r: (0, 0)),
            pl.BlockSpec((1, MAXB), lambda i, r: (0, 0)),
            pl.BlockSpec((1, MAXB), lambda i, r: (0, 0)),
        ],
        scratch_shapes=[pltpu.VMEM((1, E), jnp.float32)],
    )
    return pl.pallas_call(
        body,
        grid_spec=grid_spec,
        out_shape=[
            jax.ShapeDtypeStruct((B, 1), jnp.int32),
            jax.ShapeDtypeStruct((B, 1), jnp.int32),
            jax.ShapeDtypeStruct((B, 1), jnp.float32),
            jax.ShapeDtypeStruct((B, 1), jnp.float32),
            jax.ShapeDtypeStruct((1, E), jnp.float32),
            jax.ShapeDtypeStruct((1, MAXB), jnp.int32),
            jax.ShapeDtypeStruct((1, MAXB), jnp.int32),
            jax.ShapeDtypeStruct((1, MAXB), jnp.int32),
        ],
    )(ridx, z, centroids, tau_raw.reshape(R, 1))


def _sc_dispatch(zq, d0, d1, NROWS):
    B, W = zq.shape
    info = plsc.get_sparse_core_info()
    NC, NS = info.num_cores, info.num_subcores
    NW = NC * NS
    per_w = B // NW
    mesh = plsc.VectorSubcoreMesh(core_axis_name="c", subcore_axis_name="s")

    @functools.partial(
        pl.kernel, mesh=mesh,
        out_type=jax.ShapeDtypeStruct((NROWS, W), jnp.int32),
        scratch_types=[
            pltpu.VMEM((per_w, W), jnp.int32),
            pltpu.VMEM((per_w,), jnp.int32),
            pltpu.VMEM((per_w,), jnp.int32),
            pltpu.SemaphoreType.DMA,
        ],
    )
    def k(z_hbm, d0_hbm, d1_hbm, xs_hbm, rows_v, i0_v, i1_v, sem):
        wid = lax.axis_index("s") * NC + lax.axis_index("c")
        base = wid * per_w
        pltpu.sync_copy(z_hbm.at[pl.ds(base, per_w)], rows_v)
        pltpu.sync_copy(d0_hbm.at[pl.ds(base, per_w)], i0_v)
        pltpu.sync_copy(d1_hbm.at[pl.ds(base, per_w)], i1_v)
        pltpu.async_copy(rows_v, xs_hbm.at[i0_v], sem).wait()
        pltpu.async_copy(rows_v, xs_hbm.at[i1_v], sem).wait()

    return k(zq, d0, d1)


def _expert_body(be_ref, br_ref, ba_ref, xs_ref, w1_ref, b1_ref,
                 w2_ref, b2_ref, os_ref):
    b = pl.program_id(0)

    @pl.when(ba_ref[b] == 1)
    def _():
        h = lax.dot_general(xs_ref[...], w1_ref[0], (((1,), (0,)), ((), ())),
                            preferred_element_type=jnp.float32)
        h = _gelu_exact(h + b1_ref[0])
        o = lax.dot_general(h.astype(jnp.bfloat16), w2_ref[0],
                            (((1,), (0,)), ((), ())),
                            preferred_element_type=jnp.float32)
        os_ref[...] = (o + b2_ref[0]).astype(jnp.bfloat16)


def _run_experts(xs_bf, W1bf, b1, W2bf, b2, bexp, brow, bact,
                 E, D, H, NROWS, MAXB):
    grid_spec = pltpu.PrefetchScalarGridSpec(
        num_scalar_prefetch=3,
        grid=(MAXB,),
        in_specs=[
            pl.BlockSpec((BLK, D), lambda b, be, br, ba: (br[b], 0)),
            pl.BlockSpec((1, D, H), lambda b, be, br, ba: (be[b], 0, 0)),
            pl.BlockSpec((1, 1, H), lambda b, be, br, ba: (be[b], 0, 0)),
            pl.BlockSpec((1, H, D), lambda b, be, br, ba: (be[b], 0, 0)),
            pl.BlockSpec((1, 1, D), lambda b, be, br, ba: (be[b], 0, 0)),
        ],
        out_specs=pl.BlockSpec((BLK, D), lambda b, be, br, ba: (br[b], 0)),
    )
    return pl.pallas_call(
        _expert_body,
        grid_spec=grid_spec,
        out_shape=jax.ShapeDtypeStruct((NROWS, D), jnp.bfloat16),
    )(bexp, brow, bact, xs_bf, W1bf, b1, W2bf, b2)


def _sc_gather(osq, d0, d1):
    B = d0.shape[0]
    W = osq.shape[1]
    info = plsc.get_sparse_core_info()
    NC, NS = info.num_cores, info.num_subcores
    NW = NC * NS
    per_w = B // NW
    mesh = plsc.VectorSubcoreMesh(core_axis_name="c", subcore_axis_name="s")

    @functools.partial(
        pl.kernel, mesh=mesh,
        out_type=[jax.ShapeDtypeStruct((B, W), jnp.int32),
                  jax.ShapeDtypeStruct((B, W), jnp.int32)],
        scratch_types=[
            pltpu.VMEM((per_w, W), jnp.int32),
            pltpu.VMEM((per_w,), jnp.int32),
            pltpu.SemaphoreType.DMA,
        ],
    )
    def k(os_hbm, d0_hbm, d1_hbm, g0_hbm, g1_hbm, rows_v, idx_v, sem):
        wid = lax.axis_index("s") * NC + lax.axis_index("c")
        base = wid * per_w
        pltpu.sync_copy(d0_hbm.at[pl.ds(base, per_w)], idx_v)
        pltpu.async_copy(os_hbm.at[idx_v], rows_v, sem).wait()
        pltpu.sync_copy(rows_v, g0_hbm.at[pl.ds(base, per_w)])
        pltpu.sync_copy(d1_hbm.at[pl.ds(base, per_w)], idx_v)
        pltpu.async_copy(os_hbm.at[idx_v], rows_v, sem).wait()
        pltpu.sync_copy(rows_v, g1_hbm.at[pl.ds(base, per_w)])

    return k(osq, d0, d1)


def _combine_body(z_ref, g0_ref, g1_ref, w0_ref, w1_ref, gamma_ref,
                  beta_ref, out_ref):
    zb = z_ref[...]
    y = zb + w0_ref[...] * g0_ref[...].astype(jnp.float32) \
           + w1_ref[...] * g1_ref[...].astype(jnp.float32)
    mean = jnp.mean(y, axis=1, keepdims=True)
    yc = y - mean
    var = jnp.mean(yc * yc, axis=1, keepdims=True)
    out_ref[...] = yc * lax.rsqrt(var + 1e-5) * gamma_ref[...] + beta_ref[...]


def _run_combine(z, g0bf, g1bf, rw0, rw1, gamma, beta, B, D):
    NB = B // TB
    return pl.pallas_call(
        _combine_body,
        grid=(NB,),
        in_specs=[
            pl.BlockSpec((TB, D), lambda i: (i, 0)),
            pl.BlockSpec((TB, D), lambda i: (i, 0)),
            pl.BlockSpec((TB, D), lambda i: (i, 0)),
            pl.BlockSpec((TB, 1), lambda i: (i, 0)),
            pl.BlockSpec((TB, 1), lambda i: (i, 0)),
            pl.BlockSpec((1, D), lambda i: (0, 0)),
            pl.BlockSpec((1, D), lambda i: (0, 0)),
        ],
        out_specs=pl.BlockSpec((TB, D), lambda i: (i, 0)),
        out_shape=jax.ShapeDtypeStruct((B, D), jnp.float32),
    )(z, g0bf, g1bf, rw0, rw1, gamma.reshape(1, D), beta.reshape(1, D))


def kernel(z, router_idx, W1, b1, W2, b2, centroids, tau_raw, gamma, beta):
    B, D = z.shape
    E, _, H = W1.shape
    R = centroids.shape[0]
    CAP = B
    RPB = CAP // BLK
    NROWS = E * CAP + BLK
    MAXB = (B * 2) // BLK + E

    d0, d1, rw0, rw1, ew, bexp, brow, bact = _run_router(
        z, router_idx, centroids, tau_raw, E, B, D, R, CAP, RPB, MAXB)

    zq = lax.bitcast_convert_type(
        z.astype(jnp.bfloat16).reshape(B, D // 2, 2), jnp.int32)
    xs = _sc_dispatch(zq, d0.reshape(B), d1.reshape(B), NROWS)
    xs_bf = lax.bitcast_convert_type(xs, jnp.bfloat16).reshape(NROWS, D)

    os_bf = _run_experts(xs_bf, W1.astype(jnp.bfloat16), b1.reshape(E, 1, H),
                         W2.astype(jnp.bfloat16), b2.reshape(E, 1, D),
                         bexp.reshape(MAXB), brow.reshape(MAXB),
                         bact.reshape(MAXB), E, D, H, NROWS, MAXB)
    osq = lax.bitcast_convert_type(os_bf.reshape(NROWS, D // 2, 2), jnp.int32)

    g0, g1 = _sc_gather(osq, d0.reshape(B), d1.reshape(B))
    g0bf = lax.bitcast_convert_type(g0, jnp.bfloat16).reshape(B, D)
    g1bf = lax.bitcast_convert_type(g1, jnp.bfloat16).reshape(B, D)

    y_moe = _run_combine(z, g0bf, g1bf, rw0, rw1, gamma, beta, B, D)
    return y_moe, ew.reshape(E)
